# baseline (speedup 1.0000x reference)
import jax
import jax.numpy as jnp
from jax import lax
from jax.experimental import pallas as pl
from jax.experimental.pallas import tpu as pltpu
from jax.experimental.pallas import tpu_sc as plsc

_B, _F, _C, _N = 128, 512, 3, 66
_G = (_C * _N) // 2
_GS = 48
_GT = _G - _GS
_RC = 8
_NCHUNK = _B // _RC
_UNITS = _GS * _NCHUNK
_NW = 32
_NB = 8
_PF = 3
_K = -(-(_UNITS // _NW) // _NB) * _NB


def _pair_slab_body(x_hbm, o_hbm, *scr):
    ev = scr[0:8]
    od = scr[8:16]
    se = scr[16:24]
    sd = scr[24:32]
    so = scr[32:40]
    wid = lax.axis_index("s") * 2 + lax.axis_index("c")

    def unit_coords(k):
        u = wid + k * _NW
        g = lax.shift_right_logical(u, 4)
        r0 = lax.bitwise_and(u, 15) * _RC
        return u, g, r0

    def in_even(k, s):
        _, g, r0 = unit_coords(k)
        return pltpu.make_async_copy(
            x_hbm.at[g, 0, pl.ds(r0, _RC), :], ev[s], se[s])

    def in_odd(k, s):
        _, g, r0 = unit_coords(k)
        return pltpu.make_async_copy(
            x_hbm.at[g, 1, pl.ds(r0, _RC), :], od[s], sd[s])

    def out_copy(k, s):
        _, g, r0 = unit_coords(k)
        return pltpu.make_async_copy(
            ev[s], o_hbm.at[g, pl.ds(r0, _RC), :], so[s])

    def accumulate(od_b, ev_b):
        def row(r, carry):
            for c in range(_F // 16):
                sl = pl.ds(c * 16, 16)
                plsc.addupdate(ev_b.at[r, sl], od_b[r, sl])
            return carry
        lax.fori_loop(0, _RC, row, 0)

    for k0 in range(_PF):
        in_even(k0, k0).start()
        in_odd(k0, k0).start()

    def step_block(p, carry):
        for b in range(_NB):
            k = p * _NB + b
            u = wid + k * _NW
            valid = u < _UNITS
            pf = u + _PF * _NW < _UNITS
            kp = k + _PF
            kd = lax.max(k - (_NB - _PF), 0)

            @pl.when(valid)
            def _wait_in():
                in_even(k, b).wait()
                in_odd(k, b).wait()

            @pl.when((k >= _NB - _PF) & pf)
            def _drain_out():
                out_copy(kd, (b + _PF) % _NB).wait()

            @pl.when(pf)
            def _prefetch():
                in_even(kp, (b + _PF) % _NB).start()
                in_odd(kp, (b + _PF) % _NB).start()

            @pl.when(valid)
            def _go():
                accumulate(od[b], ev[b])
                out_copy(k, b).start()
        return carry

    lax.fori_loop(0, _K // _NB, step_block, 0)

    for m in range(max(0, _UNITS // _NW - _NB - 1), -(-_UNITS // _NW)):
        u_m = wid + m * _NW

        @pl.when((u_m + _NB * _NW >= _UNITS) & (u_m < _UNITS))
        def _final_drain():
            out_copy(m, m % _NB).wait()


_pair_slab_sc = pl.kernel(
    _pair_slab_body,
    out_type=jax.ShapeDtypeStruct((_GS, _B, _F), jnp.float32),
    mesh=plsc.VectorSubcoreMesh(core_axis_name="c", subcore_axis_name="s"),
    compiler_params=pltpu.CompilerParams(
        needs_layout_passes=False, skip_device_barrier=True),
    scratch_types=(
        [pltpu.VMEM((_RC, _F), jnp.float32) for _ in range(16)]
        + [pltpu.SemaphoreType.DMA for _ in range(24)]
    ),
)


def _pair_slab_tc_body(x_ref, o_ref):
    o_ref[0] = x_ref[0, 0] + x_ref[0, 1]


_pair_slab_tc = pl.pallas_call(
    _pair_slab_tc_body,
    grid=(_GT,),
    in_specs=[pl.BlockSpec((1, 2, _B, _F), lambda i: (_GS + i, 0, 0, 0))],
    out_specs=pl.BlockSpec((1, _B, _F), lambda i: (i, 0, 0)),
    out_shape=jax.ShapeDtypeStruct((_GT, _B, _F), jnp.float32),
)


def kernel(x):
    xt = x.transpose(2, 3, 0, 1).reshape(_G, 2, _B, _F)
    out_sc = _pair_slab_sc(xt)
    out_tc = _pair_slab_tc(xt)
    out = jnp.concatenate([out_sc, out_tc], axis=0)
    return out.reshape(_C, _N // 2, _B, _F).transpose(2, 3, 0, 1)


# speedup vs baseline: 1.5159x; 1.5159x over previous
import jax
import jax.numpy as jnp
from jax import lax
from jax.experimental import pallas as pl
from jax.experimental.pallas import tpu as pltpu
from jax.experimental.pallas import tpu_sc as plsc

_B, _F, _C, _N = 128, 512, 3, 66
_G = (_C * _N) // 2
_RC = 8
_NCHUNK = _B // _RC
_UNITS = _G * _NCHUNK
_NW = 32
_NB = 8
_PF = 3
_K = -(-(-(-_UNITS // _NW)) // _NB) * _NB


def _pair_slab_body(x_hbm, o_hbm, *scr):
    bufs = scr[0:8]
    si = scr[8:16]
    so = scr[16:24]
    wid = lax.axis_index("s") * 2 + lax.axis_index("c")

    def unit_coords(k):
        u = wid + k * _NW
        g = lax.shift_right_logical(u, 4)
        r0 = lax.bitwise_and(u, 15) * _RC
        return u, g, r0

    def in_copy(k, s):
        _, g, r0 = unit_coords(k)
        return pltpu.make_async_copy(
            x_hbm.at[g, :, pl.ds(r0, _RC), :], bufs[s], si[s])

    def out_copy(k, s):
        _, g, r0 = unit_coords(k)
        return pltpu.make_async_copy(
            bufs[s].at[1], o_hbm.at[g, pl.ds(r0, _RC), :], so[s])

    def accumulate(buf):
        def row(r, carry):
            for c in range(_F // 16):
                sl = pl.ds(c * 16, 16)
                plsc.addupdate(buf.at[1, r, sl], buf[0, r, sl])
            return carry
        lax.fori_loop(0, _RC, row, 0)

    for k0 in range(_PF):
        in_copy(k0, k0).start()

    def step_block(p, carry):
        for b in range(_NB):
            k = p * _NB + b
            u = wid + k * _NW
            valid = u < _UNITS
            pf = u + _PF * _NW < _UNITS
            kp = k + _PF
            kd = lax.max(k - (_NB - _PF), 0)

            @pl.when(valid)
            def _wait_in():
                in_copy(k, b).wait()

            @pl.when((k >= _NB - _PF) & pf)
            def _drain_out():
                out_copy(kd, (b + _PF) % _NB).wait()

            @pl.when(pf)
            def _prefetch():
                in_copy(kp, (b + _PF) % _NB).start()

            @pl.when(valid)
            def _go():
                accumulate(bufs[b])
                out_copy(k, b).start()
        return carry

    lax.fori_loop(0, _K // _NB, step_block, 0)

    for m in range(max(0, _UNITS // _NW - _NB - 1), -(-_UNITS // _NW)):
        u_m = wid + m * _NW

        @pl.when((u_m + _NB * _NW >= _UNITS) & (u_m < _UNITS))
        def _final_drain():
            out_copy(m, m % _NB).wait()


_pair_slab = pl.kernel(
    _pair_slab_body,
    out_type=jax.ShapeDtypeStruct((_G, _B, _F), jnp.float32),
    mesh=plsc.VectorSubcoreMesh(core_axis_name="c", subcore_axis_name="s"),
    compiler_params=pltpu.CompilerParams(
        needs_layout_passes=False, skip_device_barrier=True),
    scratch_types=(
        [pltpu.VMEM((2, _RC, _F), jnp.float32) for _ in range(8)]
        + [pltpu.SemaphoreType.DMA for _ in range(16)]
    ),
)


def kernel(x):
    xt = x.transpose(2, 3, 0, 1).reshape(_G, 2, _B, _F)
    out = _pair_slab(xt)
    return out.reshape(_C, _N // 2, _B, _F).transpose(2, 3, 0, 1)


# speedup vs baseline: 1.5495x; 1.0222x over previous
import jax
import jax.numpy as jnp
from jax import lax
from jax.experimental import pallas as pl
from jax.experimental.pallas import tpu as pltpu
from jax.experimental.pallas import tpu_sc as plsc

_B, _F, _C, _N = 128, 512, 3, 66
_G = (_C * _N) // 2
_RC = 8
_NCHUNK = _B // _RC
_UNITS = _G * _NCHUNK
_NW = 32
_NBI = 4
_NBO = 4
_K = 52


def _pair_slab_body(x_hbm, o_hbm, *scr):
    ins = tuple(zip(scr[0:4], scr[8:12]))
    outs = tuple(zip(scr[4:8], scr[12:16]))
    wid = lax.axis_index("s") * 2 + lax.axis_index("c")

    def unit_coords(k):
        u = wid + k * _NW
        g = lax.shift_right_logical(u, 4)
        r0 = lax.bitwise_and(u, 15) * _RC
        return u, g, r0

    def in_copy(k, slot):
        _, g, r0 = unit_coords(k)
        buf, sem = ins[slot]
        return pltpu.make_async_copy(
            x_hbm.at[g, :, pl.ds(r0, _RC), :], buf, sem)

    def out_copy(k, slot):
        _, g, r0 = unit_coords(k)
        buf, sem = outs[slot]
        return pltpu.make_async_copy(
            buf, o_hbm.at[g, pl.ds(r0, _RC), :], sem)

    def compute(in_b, out_b):
        def row(r, carry):
            for c in range(_F // 16):
                sl = pl.ds(c * 16, 16)
                out_b[r, sl] = in_b[0, r, sl] + in_b[1, r, sl]
            return carry
        lax.fori_loop(0, _RC, row, 0)

    for k0 in range(_NBI - 1):
        in_copy(k0, k0).start()

    def quad(p, carry):
        for b in range(_NBI):
            k = p * _NBI + b
            u = wid + k * _NW
            valid = u < _UNITS
            kw = lax.max(k - _NBO, 0)

            @pl.when(valid)
            def _wait_in():
                in_copy(k, b).wait()

            @pl.when((k >= _NBO) & (u - _NBO * _NW < _UNITS))
            def _wait_out():
                out_copy(kw, b % _NBO).wait()

            @pl.when(u + (_NBI - 1) * _NW < _UNITS)
            def _prefetch():
                in_copy(k + _NBI - 1, (b + _NBI - 1) % _NBI).start()

            @pl.when(valid)
            def _go():
                compute(ins[b][0], outs[b % _NBO][0])
                out_copy(k, b % _NBO).start()
        return carry

    lax.fori_loop(0, _K // _NBI, quad, 0)

    for m in range(_K - _NBO, _K):
        u_m = wid + m * _NW

        @pl.when(u_m < _UNITS)
        def _final_drain():
            out_copy(m, m % _NBO).wait()


_pair_slab = pl.kernel(
    _pair_slab_body,
    out_type=jax.ShapeDtypeStruct((_G, _B, _F), jnp.float32),
    mesh=plsc.VectorSubcoreMesh(core_axis_name="c", subcore_axis_name="s"),
    compiler_params=pltpu.CompilerParams(
        needs_layout_passes=False, skip_device_barrier=True),
    scratch_types=(
        [pltpu.VMEM((2, _RC, _F), jnp.float32) for _ in range(4)]
        + [pltpu.VMEM((_RC, _F), jnp.float32) for _ in range(4)]
        + [pltpu.SemaphoreType.DMA for _ in range(8)]
    ),
)


def kernel(x):
    xt = x.transpose(2, 3, 0, 1).reshape(_G, 2, _B, _F)
    out = _pair_slab(xt)
    return out.reshape(_C, _N // 2, _B, _F).transpose(2, 3, 0, 1)
